# Initial kernel scaffold; baseline (speedup 1.0000x reference)
import jax
import jax.numpy as jnp
from jax.experimental import pallas as pl


def kernel(x, edge_index, W_gcn, b_gcn, Wl1, bl1, Wr1, Wl2, bl2, Wr2, W0, b0, g0, be0, W1, b1, g1, be1, W2, b2, g2, be2, W3, b3):
    raise NotImplementedError("write your pallas kernel here")


# speedup vs baseline: 20.8509x; 20.8509x over previous
import functools

import jax
import jax.numpy as jnp
from jax import lax
from jax.experimental import pallas as pl
from jax.experimental.pallas import tpu as pltpu
from jax.experimental.pallas import tpu_sc as plsc

N_NODES = 10000
FDIM = 128
NC = 2
NS = 16
NW = NC * NS
NPAD = 10240
TRASH = NPAD - N_NODES
ECHUNK = 128
NCHUNK = 80
EPT = NCHUNK * ECHUNK
EPAD = NW * EPT
RPT = NPAD // NS
BLK = 1024
GRID = NPAD // BLK

_mesh = plsc.VectorSubcoreMesh(core_axis_name="c", subcore_axis_name="s")
_sc_params = pltpu.CompilerParams(needs_layout_passes=False)


def _count_body(dst_hbm, out_hbm, dst_v, ones_v, zrow_v, cnt_sh):
    c = lax.axis_index("c")
    s = lax.axis_index("s")
    wid = c * NS + s

    def _fill_ones(i, carry):
        ones_v[pl.ds(i * 16, 16)] = jnp.ones((16,), jnp.float32)
        return carry

    lax.fori_loop(0, ECHUNK // 16, _fill_ones, 0)

    def _fill_zero(i, carry):
        zrow_v[pl.ds(i * 16, 16)] = jnp.zeros((16,), jnp.float32)
        return carry

    lax.fori_loop(0, RPT // 16, _fill_zero, 0)
    pltpu.sync_copy(zrow_v, cnt_sh.at[pl.ds(s * RPT, RPT)])
    pltpu.sync_copy(dst_hbm.at[wid], dst_v)
    plsc.subcore_barrier()

    def _step(j, carry):
        pltpu.sync_copy(ones_v, cnt_sh.at[dst_v.at[j]], add=True)
        return carry

    lax.fori_loop(0, NCHUNK, _step, 0)
    plsc.subcore_barrier()
    pltpu.sync_copy(cnt_sh.at[pl.ds(s * RPT, RPT)],
                    out_hbm.at[c, pl.ds(s * RPT, RPT)])


_count_call = pl.kernel(
    _count_body,
    out_type=jax.ShapeDtypeStruct((NC, NPAD), jnp.float32),
    mesh=_mesh,
    scratch_types=[
        pltpu.VMEM((NCHUNK, ECHUNK), jnp.int32),
        pltpu.VMEM((ECHUNK,), jnp.float32),
        pltpu.VMEM((RPT,), jnp.float32),
        pltpu.VMEM_SHARED((NPAD,), jnp.float32),
    ],
)


def _scatter_body(coef, *refs):
    if coef:
        (table_hbm, src_hbm, dst_hbm, w_hbm, part_out, cpart_out,
         src_v, dst_v, rows, sem, w_v, wrow, acc_sh, c_sh) = refs
    else:
        (table_hbm, src_hbm, dst_hbm, part_out,
         src_v, dst_v, rows, sem, acc_sh) = refs
    c = lax.axis_index("c")
    s = lax.axis_index("s")
    wid = c * NS + s

    def _zrow(r, carry):
        for k in range(FDIM // 16):
            rows[r, pl.ds(k * 16, 16)] = jnp.zeros((16,), jnp.float32)
        return carry

    lax.fori_loop(0, ECHUNK, _zrow, 0)
    for t in range(RPT // ECHUNK):
        pltpu.sync_copy(rows, acc_sh.at[pl.ds(s * RPT + t * ECHUNK, ECHUNK)])

    pltpu.sync_copy(src_hbm.at[wid], src_v)
    pltpu.sync_copy(dst_hbm.at[wid], dst_v)
    if coef:
        pltpu.sync_copy(w_hbm, w_v)

        def _zw(i, carry):
            wrow[pl.ds(i * 16, 16)] = jnp.zeros((16,), jnp.float32)
            return carry

        lax.fori_loop(0, ECHUNK // 16, _zw, 0)
        for t in range(RPT // ECHUNK):
            pltpu.sync_copy(wrow, c_sh.at[pl.ds(s * RPT + t * ECHUNK, ECHUNK)])
    plsc.subcore_barrier()

    def _step(j, carry):
        pltpu.async_copy(table_hbm.at[src_v.at[j]], rows, sem).wait()
        if coef:
            for k in range(ECHUNK // 16):
                di = dst_v[j, pl.ds(k * 16, 16)]
                wv = plsc.load_gather(w_v, [di])
                wrow[pl.ds(k * 16, 16)] = wv
            pltpu.sync_copy(wrow, c_sh.at[src_v.at[j]], add=True)
        pltpu.sync_copy(rows, acc_sh.at[dst_v.at[j]], add=True)
        return carry

    lax.fori_loop(0, NCHUNK, _step, 0)
    plsc.subcore_barrier()
    pltpu.sync_copy(acc_sh.at[pl.ds(s * RPT, RPT)],
                    part_out.at[c, pl.ds(s * RPT, RPT)])
    if coef:
        pltpu.sync_copy(c_sh.at[pl.ds(s * RPT, RPT)],
                        cpart_out.at[c, pl.ds(s * RPT, RPT)])


def _make_scatter(coef):
    part_type = jax.ShapeDtypeStruct((NC, NPAD, FDIM), jnp.float32)
    out_type = [part_type]
    scratch = [
        pltpu.VMEM((NCHUNK, ECHUNK), jnp.int32),
        pltpu.VMEM((NCHUNK, ECHUNK), jnp.int32),
        pltpu.VMEM((ECHUNK, FDIM), jnp.float32),
        pltpu.SemaphoreType.DMA,
    ]
    if coef:
        out_type.append(jax.ShapeDtypeStruct((NC, NPAD), jnp.float32))
        scratch += [
            pltpu.VMEM((NPAD,), jnp.float32),
            pltpu.VMEM((ECHUNK,), jnp.float32),
        ]
    scratch.append(pltpu.VMEM_SHARED((NPAD, FDIM), jnp.float32))
    if coef:
        scratch.append(pltpu.VMEM_SHARED((NPAD,), jnp.float32))
    return pl.kernel(
        functools.partial(_scatter_body, coef),
        out_type=out_type if coef else part_type,
        mesh=_mesh,
        scratch_types=scratch,
        compiler_params=_sc_params,
    )


_scatter_coef_call = _make_scatter(True)
_scatter_call = _make_scatter(False)


def _k2_body(x_ref, wg_ref, cntp_ref, y_ref, dinv_ref, wcol_ref):
    i = pl.program_id(0)
    cnt = cntp_ref[0] + cntp_ref[1]
    dinv = lax.rsqrt(cnt + 1.0)
    rows = i * BLK + lax.broadcasted_iota(jnp.int32, (BLK, 1), 0)
    wv = jnp.where(rows < N_NODES, 1.0 / jnp.maximum(cnt, 1.0), 0.0)
    xw = jnp.dot(x_ref[...], wg_ref[...], preferred_element_type=jnp.float32)
    y_ref[...] = xw * dinv
    dinv_ref[...] = dinv
    wcol_ref[...] = wv


def _k2_call(x_pad, W_gcn, cntp3):
    return pl.pallas_call(
        _k2_body,
        grid=(GRID,),
        in_specs=[
            pl.BlockSpec((BLK, FDIM), lambda i: (i, 0)),
            pl.BlockSpec((FDIM, FDIM), lambda i: (0, 0)),
            pl.BlockSpec((NC, BLK, 1), lambda i: (0, i, 0)),
        ],
        out_specs=[
            pl.BlockSpec((BLK, FDIM), lambda i: (i, 0)),
            pl.BlockSpec((BLK, 1), lambda i: (i, 0)),
            pl.BlockSpec((BLK, 1), lambda i: (i, 0)),
        ],
        out_shape=[
            jax.ShapeDtypeStruct((NPAD, FDIM), jnp.float32),
            jax.ShapeDtypeStruct((NPAD, 1), jnp.float32),
            jax.ShapeDtypeStruct((NPAD, 1), jnp.float32),
        ],
    )(x_pad, W_gcn, cntp3)


def _k4_body(p_ref, y_ref, dinv_ref, b_ref, h0_ref):
    t = (p_ref[0] + p_ref[1] + y_ref[...]) * dinv_ref[...]
    h0_ref[...] = jnp.maximum(t + b_ref[...], 0.0)


def _k4_call(parts, y, dinv_col, b_gcn2):
    return pl.pallas_call(
        _k4_body,
        grid=(GRID,),
        in_specs=[
            pl.BlockSpec((NC, BLK, FDIM), lambda i: (0, i, 0)),
            pl.BlockSpec((BLK, FDIM), lambda i: (i, 0)),
            pl.BlockSpec((BLK, 1), lambda i: (i, 0)),
            pl.BlockSpec((1, FDIM), lambda i: (0, 0)),
        ],
        out_specs=pl.BlockSpec((BLK, FDIM), lambda i: (i, 0)),
        out_shape=jax.ShapeDtypeStruct((NPAD, FDIM), jnp.float32),
    )(parts, y, dinv_col, b_gcn2)


def _k6_body(q_ref, h0_ref, wcol_ref, cpart_ref, wl_ref, bl_ref, wr_ref,
             s_ref):
    i = pl.program_id(0)
    agg = (q_ref[0] + q_ref[1]) * wcol_ref[...]
    h1 = (jnp.dot(agg, wl_ref[...], preferred_element_type=jnp.float32)
          + bl_ref[...]
          + jnp.dot(h0_ref[...], wr_ref[...],
                    preferred_element_type=jnp.float32))
    h1 = jnp.maximum(h1, 0.0)
    rows = i * BLK + lax.broadcasted_iota(jnp.int32, (BLK, 1), 0)
    h1m = jnp.where(rows < N_NODES, h1, 0.0)
    ccol = cpart_ref[0] + cpart_ref[1]
    s0 = jnp.sum(h1m, axis=0, keepdims=True)
    s1 = jnp.sum(ccol * h1m, axis=0, keepdims=True)
    blk = jnp.concatenate([s0, s1], axis=0)

    @pl.when(i == 0)
    def _():
        s_ref[...] = blk

    @pl.when(i > 0)
    def _():
        s_ref[...] += blk


def _k6_call(parts2, h0, w_col, cpart3, Wl1, bl1_2, Wr1):
    return pl.pallas_call(
        _k6_body,
        grid=(GRID,),
        in_specs=[
            pl.BlockSpec((NC, BLK, FDIM), lambda i: (0, i, 0)),
            pl.BlockSpec((BLK, FDIM), lambda i: (i, 0)),
            pl.BlockSpec((BLK, 1), lambda i: (i, 0)),
            pl.BlockSpec((NC, BLK, 1), lambda i: (0, i, 0)),
            pl.BlockSpec((FDIM, FDIM), lambda i: (0, 0)),
            pl.BlockSpec((1, FDIM), lambda i: (0, 0)),
            pl.BlockSpec((FDIM, FDIM), lambda i: (0, 0)),
        ],
        out_specs=pl.BlockSpec((2, FDIM), lambda i: (0, 0)),
        out_shape=jax.ShapeDtypeStruct((2, FDIM), jnp.float32),
    )(parts2, h0, w_col, cpart3, Wl1, bl1_2, Wr1)


def _k7_body(s_ref, wl2, bl2, wr2, w0, b0, g0, be0, w1, b1, g1, be1,
             w2, b2, g2, be2, w3, b3, out_ref):
    inv = 1.0 / jnp.sqrt(jnp.float32(1.0 + 1e-5))
    m_h1 = s_ref[0:1, :] * (1.0 / N_NODES)
    m_agg = s_ref[1:2, :] * (1.0 / N_NODES)
    p = (jnp.dot(m_agg, wl2[...], preferred_element_type=jnp.float32)
         + bl2[...]
         + jnp.dot(m_h1, wr2[...], preferred_element_type=jnp.float32))
    z = jnp.dot(p, w0[...], preferred_element_type=jnp.float32) + b0[...]
    z = jnp.tanh(z * inv * g0[...] + be0[...])
    z = jnp.dot(z, w1[...], preferred_element_type=jnp.float32) + b1[...]
    z = jnp.tanh(z * inv * g1[...] + be1[...])
    z = jnp.dot(z, w2[...], preferred_element_type=jnp.float32) + b2[...]
    z = jnp.tanh(z * inv * g2[...] + be2[...])
    z = jnp.dot(z, w3[...], preferred_element_type=jnp.float32) + b3[...]
    z = z - jnp.max(z, axis=1, keepdims=True)
    ez = jnp.exp(z)
    out_ref[...] = ez / jnp.sum(ez, axis=1, keepdims=True)


def _k7_call(S, *weights):
    return pl.pallas_call(
        _k7_body,
        out_shape=jax.ShapeDtypeStruct((1, 10), jnp.float32),
    )(S, *weights)


def kernel(x, edge_index, W_gcn, b_gcn, Wl1, bl1, Wr1, Wl2, bl2, Wr2,
           W0, b0, g0, be0, W1, b1, g1, be1, W2, b2, g2, be2, W3, b3):
    f32 = jnp.float32
    n, d = x.shape
    x_pad = jnp.concatenate([x, jnp.zeros((NPAD - n, d), f32)], axis=0)

    src = edge_index[0]
    dst = edge_index[1]
    npad_e = EPAD - src.shape[0]
    pi = jnp.arange(npad_e, dtype=jnp.int32)
    src_p = jnp.concatenate([src, pi % N_NODES])
    dst_p = jnp.concatenate([dst, N_NODES + pi % TRASH])
    src3 = src_p.reshape(NW, NCHUNK, ECHUNK)
    dst3 = dst_p.reshape(NW, NCHUNK, ECHUNK)

    cntp = _count_call(dst3)
    y, dinv_col, w_col = _k2_call(x_pad, W_gcn, cntp.reshape(NC, NPAD, 1))
    parts, cpart = _scatter_coef_call(y, src3, dst3, w_col.reshape(NPAD))
    h0 = _k4_call(parts, y, dinv_col, b_gcn.reshape(1, FDIM))
    parts2 = _scatter_call(h0, src3, dst3)
    S = _k6_call(parts2, h0, w_col, cpart.reshape(NC, NPAD, 1),
                 Wl1, bl1.reshape(1, FDIM), Wr1)
    return _k7_call(S, Wl2, bl2.reshape(1, FDIM), Wr2,
                    W0, b0.reshape(1, 200), g0.reshape(1, 200),
                    be0.reshape(1, 200),
                    W1, b1.reshape(1, 100), g1.reshape(1, 100),
                    be1.reshape(1, 100),
                    W2, b2.reshape(1, 50), g2.reshape(1, 50),
                    be2.reshape(1, 50),
                    W3, b3.reshape(1, 10))


# speedup vs baseline: 29.6007x; 1.4196x over previous
import functools

import jax
import jax.numpy as jnp
from jax import lax
from jax.experimental import pallas as pl
from jax.experimental.pallas import tpu as pltpu
from jax.experimental.pallas import tpu_sc as plsc

N_NODES = 10000
FDIM = 128
NC = 2
NS = 16
NW = NC * NS
NPAD = 10240
TRASH = NPAD - N_NODES
ECHUNK = 128
NCHUNK = 80
EPT = NCHUNK * ECHUNK
EPAD = NW * EPT
RPT = NPAD // NS
BLK = 1024
GRID = NPAD // BLK

_mesh = plsc.VectorSubcoreMesh(core_axis_name="c", subcore_axis_name="s")
_sc_params = pltpu.CompilerParams(needs_layout_passes=False)


def _count_body(dst_hbm, out_hbm, dst_v, ones_v, zrow_v, cnt_sh):
    c = lax.axis_index("c")
    s = lax.axis_index("s")
    wid = c * NS + s

    def _fill_ones(i, carry):
        ones_v[pl.ds(i * 16, 16)] = jnp.ones((16,), jnp.float32)
        return carry

    lax.fori_loop(0, ECHUNK // 16, _fill_ones, 0)

    def _fill_zero(i, carry):
        zrow_v[pl.ds(i * 16, 16)] = jnp.zeros((16,), jnp.float32)
        return carry

    lax.fori_loop(0, RPT // 16, _fill_zero, 0)
    pltpu.sync_copy(zrow_v, cnt_sh.at[pl.ds(s * RPT, RPT)])
    pltpu.sync_copy(dst_hbm.at[wid], dst_v)
    plsc.subcore_barrier()

    def _step(j, carry):
        pltpu.sync_copy(ones_v, cnt_sh.at[dst_v.at[j]], add=True)
        return carry

    lax.fori_loop(0, NCHUNK, _step, 0)
    plsc.subcore_barrier()
    pltpu.sync_copy(cnt_sh.at[pl.ds(s * RPT, RPT)],
                    out_hbm.at[c, pl.ds(s * RPT, RPT)])


_count_call = pl.kernel(
    _count_body,
    out_type=jax.ShapeDtypeStruct((NC, NPAD), jnp.float32),
    mesh=_mesh,
    scratch_types=[
        pltpu.VMEM((NCHUNK, ECHUNK), jnp.int32),
        pltpu.VMEM((ECHUNK,), jnp.float32),
        pltpu.VMEM((RPT,), jnp.float32),
        pltpu.VMEM_SHARED((NPAD,), jnp.float32),
    ],
)


HCHUNK = NCHUNK // 2


def _scatter_body(coef, *refs):
    if coef:
        (table_hbm, src_hbm, dst_hbm, w_hbm, part_out, cpart_out,
         src_v, dst_v, rows_a, rows_b, sem_a, sem_b,
         wbuf_a, wbuf_b, sem_wa, sem_wb, acc_sh, c_sh) = refs
    else:
        (table_hbm, src_hbm, dst_hbm, part_out,
         src_v, dst_v, rows_a, rows_b, sem_a, sem_b, acc_sh) = refs
    c = lax.axis_index("c")
    s = lax.axis_index("s")
    wid = c * NS + s

    def _zrow(r, carry):
        for k in range(FDIM // 16):
            rows_a[r, pl.ds(k * 16, 16)] = jnp.zeros((16,), jnp.float32)
        return carry

    lax.fori_loop(0, ECHUNK, _zrow, 0)
    for t in range(RPT // ECHUNK):
        pltpu.sync_copy(rows_a, acc_sh.at[pl.ds(s * RPT + t * ECHUNK, ECHUNK)])
    if coef:
        def _zw(i, carry):
            wbuf_a[pl.ds(i * 16, 16)] = jnp.zeros((16,), jnp.float32)
            return carry

        lax.fori_loop(0, ECHUNK // 16, _zw, 0)
        for t in range(RPT // ECHUNK):
            pltpu.sync_copy(wbuf_a,
                            c_sh.at[pl.ds(s * RPT + t * ECHUNK, ECHUNK)])
    plsc.subcore_barrier()

    def _gather(j, rbuf, rsem, wbuf, wsem):
        pltpu.async_copy(table_hbm.at[src_v.at[j]], rbuf, rsem)
        if coef:
            pltpu.async_copy(w_hbm.at[dst_v.at[j]], wbuf, wsem)

    def _drain_scatter(j, rbuf, rsem, wbuf, wsem):
        pltpu.make_async_copy(table_hbm.at[src_v.at[j]], rbuf, rsem).wait()
        if coef:
            pltpu.make_async_copy(w_hbm.at[dst_v.at[j]], wbuf, wsem).wait()
            pltpu.sync_copy(wbuf, c_sh.at[src_v.at[j]], add=True)
        pltpu.sync_copy(rbuf, acc_sh.at[dst_v.at[j]], add=True)

    for half in range(2):
        pltpu.sync_copy(src_hbm.at[wid, pl.ds(half * HCHUNK, HCHUNK)], src_v)
        pltpu.sync_copy(dst_hbm.at[wid, pl.ds(half * HCHUNK, HCHUNK)], dst_v)
        _gather(0, rows_a, sem_a,
                wbuf_a if coef else None, sem_wa if coef else None)

        def _step(j2, carry):
            j = j2 * 2
            _gather(j + 1, rows_b, sem_b,
                    wbuf_b if coef else None, sem_wb if coef else None)
            _drain_scatter(j, rows_a, sem_a,
                           wbuf_a if coef else None,
                           sem_wa if coef else None)

            @pl.when(j + 2 < HCHUNK)
            def _():
                _gather(j + 2, rows_a, sem_a,
                        wbuf_a if coef else None,
                        sem_wa if coef else None)

            _drain_scatter(j + 1, rows_b, sem_b,
                           wbuf_b if coef else None,
                           sem_wb if coef else None)
            return carry

        lax.fori_loop(0, HCHUNK // 2, _step, 0)
    plsc.subcore_barrier()
    pltpu.sync_copy(acc_sh.at[pl.ds(s * RPT, RPT)],
                    part_out.at[c, pl.ds(s * RPT, RPT)])
    if coef:
        pltpu.sync_copy(c_sh.at[pl.ds(s * RPT, RPT)],
                        cpart_out.at[c, pl.ds(s * RPT, RPT)])


def _make_scatter(coef):
    part_type = jax.ShapeDtypeStruct((NC, NPAD, FDIM), jnp.float32)
    out_type = [part_type]
    scratch = [
        pltpu.VMEM((HCHUNK, ECHUNK), jnp.int32),
        pltpu.VMEM((HCHUNK, ECHUNK), jnp.int32),
        pltpu.VMEM((ECHUNK, FDIM), jnp.float32),
        pltpu.VMEM((ECHUNK, FDIM), jnp.float32),
        pltpu.SemaphoreType.DMA,
        pltpu.SemaphoreType.DMA,
    ]
    if coef:
        out_type.append(jax.ShapeDtypeStruct((NC, NPAD), jnp.float32))
        scratch += [
            pltpu.VMEM((ECHUNK,), jnp.float32),
            pltpu.VMEM((ECHUNK,), jnp.float32),
            pltpu.SemaphoreType.DMA,
            pltpu.SemaphoreType.DMA,
        ]
    scratch.append(pltpu.VMEM_SHARED((NPAD, FDIM), jnp.float32))
    if coef:
        scratch.append(pltpu.VMEM_SHARED((NPAD,), jnp.float32))
    return pl.kernel(
        functools.partial(_scatter_body, coef),
        out_type=out_type if coef else part_type,
        mesh=_mesh,
        scratch_types=scratch,
        compiler_params=_sc_params,
    )


_scatter_coef_call = _make_scatter(True)
_scatter_call = _make_scatter(False)


def _k2_body(x_ref, wg_ref, cntp_ref, y_ref, dinv_ref, wcol_ref):
    i = pl.program_id(0)
    cnt = cntp_ref[0] + cntp_ref[1]
    dinv = lax.rsqrt(cnt + 1.0)
    rows = i * BLK + lax.broadcasted_iota(jnp.int32, (BLK, 1), 0)
    wv = jnp.where(rows < N_NODES, 1.0 / jnp.maximum(cnt, 1.0), 0.0)
    xw = jnp.dot(x_ref[...], wg_ref[...], preferred_element_type=jnp.float32)
    y_ref[...] = xw * dinv
    dinv_ref[...] = dinv
    wcol_ref[...] = wv


def _k2_call(x_pad, W_gcn, cntp3):
    return pl.pallas_call(
        _k2_body,
        grid=(GRID,),
        in_specs=[
            pl.BlockSpec((BLK, FDIM), lambda i: (i, 0)),
            pl.BlockSpec((FDIM, FDIM), lambda i: (0, 0)),
            pl.BlockSpec((NC, BLK, 1), lambda i: (0, i, 0)),
        ],
        out_specs=[
            pl.BlockSpec((BLK, FDIM), lambda i: (i, 0)),
            pl.BlockSpec((BLK, 1), lambda i: (i, 0)),
            pl.BlockSpec((BLK, 1), lambda i: (i, 0)),
        ],
        out_shape=[
            jax.ShapeDtypeStruct((NPAD, FDIM), jnp.float32),
            jax.ShapeDtypeStruct((NPAD, 1), jnp.float32),
            jax.ShapeDtypeStruct((NPAD, 1), jnp.float32),
        ],
    )(x_pad, W_gcn, cntp3)


def _k4_body(p_ref, y_ref, dinv_ref, b_ref, h0_ref):
    t = (p_ref[0] + p_ref[1] + y_ref[...]) * dinv_ref[...]
    h0_ref[...] = jnp.maximum(t + b_ref[...], 0.0)


def _k4_call(parts, y, dinv_col, b_gcn2):
    return pl.pallas_call(
        _k4_body,
        grid=(GRID,),
        in_specs=[
            pl.BlockSpec((NC, BLK, FDIM), lambda i: (0, i, 0)),
            pl.BlockSpec((BLK, FDIM), lambda i: (i, 0)),
            pl.BlockSpec((BLK, 1), lambda i: (i, 0)),
            pl.BlockSpec((1, FDIM), lambda i: (0, 0)),
        ],
        out_specs=pl.BlockSpec((BLK, FDIM), lambda i: (i, 0)),
        out_shape=jax.ShapeDtypeStruct((NPAD, FDIM), jnp.float32),
    )(parts, y, dinv_col, b_gcn2)


def _k6_body(q_ref, h0_ref, wcol_ref, cpart_ref, wl_ref, bl_ref, wr_ref,
             s_ref):
    i = pl.program_id(0)
    agg = (q_ref[0] + q_ref[1]) * wcol_ref[...]
    h1 = (jnp.dot(agg, wl_ref[...], preferred_element_type=jnp.float32)
          + bl_ref[...]
          + jnp.dot(h0_ref[...], wr_ref[...],
                    preferred_element_type=jnp.float32))
    h1 = jnp.maximum(h1, 0.0)
    rows = i * BLK + lax.broadcasted_iota(jnp.int32, (BLK, 1), 0)
    h1m = jnp.where(rows < N_NODES, h1, 0.0)
    ccol = cpart_ref[0] + cpart_ref[1]
    s0 = jnp.sum(h1m, axis=0, keepdims=True)
    s1 = jnp.sum(ccol * h1m, axis=0, keepdims=True)
    blk = jnp.concatenate([s0, s1], axis=0)

    @pl.when(i == 0)
    def _():
        s_ref[...] = blk

    @pl.when(i > 0)
    def _():
        s_ref[...] += blk


def _k6_call(parts2, h0, w_col, cpart3, Wl1, bl1_2, Wr1):
    return pl.pallas_call(
        _k6_body,
        grid=(GRID,),
        in_specs=[
            pl.BlockSpec((NC, BLK, FDIM), lambda i: (0, i, 0)),
            pl.BlockSpec((BLK, FDIM), lambda i: (i, 0)),
            pl.BlockSpec((BLK, 1), lambda i: (i, 0)),
            pl.BlockSpec((NC, BLK, 1), lambda i: (0, i, 0)),
            pl.BlockSpec((FDIM, FDIM), lambda i: (0, 0)),
            pl.BlockSpec((1, FDIM), lambda i: (0, 0)),
            pl.BlockSpec((FDIM, FDIM), lambda i: (0, 0)),
        ],
        out_specs=pl.BlockSpec((2, FDIM), lambda i: (0, 0)),
        out_shape=jax.ShapeDtypeStruct((2, FDIM), jnp.float32),
    )(parts2, h0, w_col, cpart3, Wl1, bl1_2, Wr1)


def _k7_body(s_ref, wl2, bl2, wr2, w0, b0, g0, be0, w1, b1, g1, be1,
             w2, b2, g2, be2, w3, b3, out_ref):
    inv = 1.0 / jnp.sqrt(jnp.float32(1.0 + 1e-5))
    m_h1 = s_ref[0:1, :] * (1.0 / N_NODES)
    m_agg = s_ref[1:2, :] * (1.0 / N_NODES)
    p = (jnp.dot(m_agg, wl2[...], preferred_element_type=jnp.float32)
         + bl2[...]
         + jnp.dot(m_h1, wr2[...], preferred_element_type=jnp.float32))
    z = jnp.dot(p, w0[...], preferred_element_type=jnp.float32) + b0[...]
    z = jnp.tanh(z * inv * g0[...] + be0[...])
    z = jnp.dot(z, w1[...], preferred_element_type=jnp.float32) + b1[...]
    z = jnp.tanh(z * inv * g1[...] + be1[...])
    z = jnp.dot(z, w2[...], preferred_element_type=jnp.float32) + b2[...]
    z = jnp.tanh(z * inv * g2[...] + be2[...])
    z = jnp.dot(z, w3[...], preferred_element_type=jnp.float32) + b3[...]
    z = z - jnp.max(z, axis=1, keepdims=True)
    ez = jnp.exp(z)
    out_ref[...] = ez / jnp.sum(ez, axis=1, keepdims=True)


def _k7_call(S, *weights):
    return pl.pallas_call(
        _k7_body,
        out_shape=jax.ShapeDtypeStruct((1, 10), jnp.float32),
    )(S, *weights)


def kernel(x, edge_index, W_gcn, b_gcn, Wl1, bl1, Wr1, Wl2, bl2, Wr2,
           W0, b0, g0, be0, W1, b1, g1, be1, W2, b2, g2, be2, W3, b3):
    f32 = jnp.float32
    n, d = x.shape
    x_pad = jnp.concatenate([x, jnp.zeros((NPAD - n, d), f32)], axis=0)

    src = edge_index[0]
    dst = edge_index[1]
    npad_e = EPAD - src.shape[0]
    pi = jnp.arange(npad_e, dtype=jnp.int32)
    src_p = jnp.concatenate([src, pi % N_NODES])
    dst_p = jnp.concatenate([dst, N_NODES + pi % TRASH])
    src3 = src_p.reshape(NW, NCHUNK, ECHUNK)
    dst3 = dst_p.reshape(NW, NCHUNK, ECHUNK)

    cntp = _count_call(dst3)
    y, dinv_col, w_col = _k2_call(x_pad, W_gcn, cntp.reshape(NC, NPAD, 1))
    parts, cpart = _scatter_coef_call(y, src3, dst3, w_col.reshape(NPAD))
    h0 = _k4_call(parts, y, dinv_col, b_gcn.reshape(1, FDIM))
    parts2 = _scatter_call(h0, src3, dst3)
    S = _k6_call(parts2, h0, w_col, cpart.reshape(NC, NPAD, 1),
                 Wl1, bl1.reshape(1, FDIM), Wr1)
    return _k7_call(S, Wl2, bl2.reshape(1, FDIM), Wr2,
                    W0, b0.reshape(1, 200), g0.reshape(1, 200),
                    be0.reshape(1, 200),
                    W1, b1.reshape(1, 100), g1.reshape(1, 100),
                    be1.reshape(1, 100),
                    W2, b2.reshape(1, 50), g2.reshape(1, 50),
                    be2.reshape(1, 50),
                    W3, b3.reshape(1, 10))


# speedup vs baseline: 30.0372x; 1.0147x over previous
import functools

import jax
import jax.numpy as jnp
from jax import lax
from jax.experimental import pallas as pl
from jax.experimental.pallas import tpu as pltpu
from jax.experimental.pallas import tpu_sc as plsc

N_NODES = 10000
FDIM = 128
NC = 2
NS = 16
NW = NC * NS
NPAD = 10240
TRASH = NPAD - N_NODES
ECHUNK = 128
NCHUNK = 80
EPT = NCHUNK * ECHUNK
EPAD = NW * EPT
RPT = NPAD // NS
BLK = 1024
GRID = NPAD // BLK

_mesh = plsc.VectorSubcoreMesh(core_axis_name="c", subcore_axis_name="s")
_sc_params = pltpu.CompilerParams(needs_layout_passes=False)


def _count_body(dst_hbm, out_hbm, dst_v, ones_v, zrow_v, cnt_sh):
    c = lax.axis_index("c")
    s = lax.axis_index("s")
    wid = c * NS + s

    def _fill_ones(i, carry):
        ones_v[pl.ds(i * 16, 16)] = jnp.ones((16,), jnp.float32)
        return carry

    lax.fori_loop(0, ECHUNK // 16, _fill_ones, 0)

    def _fill_zero(i, carry):
        zrow_v[pl.ds(i * 16, 16)] = jnp.zeros((16,), jnp.float32)
        return carry

    lax.fori_loop(0, RPT // 16, _fill_zero, 0)
    pltpu.sync_copy(zrow_v, cnt_sh.at[pl.ds(s * RPT, RPT)])
    pltpu.sync_copy(dst_hbm.at[wid], dst_v)
    plsc.subcore_barrier()

    def _step(j, carry):
        pltpu.sync_copy(ones_v, cnt_sh.at[dst_v.at[j]], add=True)
        return carry

    lax.fori_loop(0, NCHUNK, _step, 0)
    plsc.subcore_barrier()
    pltpu.sync_copy(cnt_sh.at[pl.ds(s * RPT, RPT)],
                    out_hbm.at[c, pl.ds(s * RPT, RPT)])


_count_call = pl.kernel(
    _count_body,
    out_type=jax.ShapeDtypeStruct((NC, NPAD), jnp.float32),
    mesh=_mesh,
    scratch_types=[
        pltpu.VMEM((NCHUNK, ECHUNK), jnp.int32),
        pltpu.VMEM((ECHUNK,), jnp.float32),
        pltpu.VMEM((RPT,), jnp.float32),
        pltpu.VMEM_SHARED((NPAD,), jnp.float32),
    ],
)


HCHUNK = NCHUNK // 2


def _scatter_body(coef, *refs):
    if coef:
        (table_hbm, src_hbm, dst_hbm, w_hbm, part_out, cpart_out,
         src_v, dst_v, rows_a, rows_b, sem_a, sem_b,
         wbuf_a, wbuf_b, sem_wa, sem_wb, acc_sh, c_sh) = refs
    else:
        (table_hbm, src_hbm, dst_hbm, part_out,
         src_v, dst_v, rows_a, rows_b, sem_a, sem_b, acc_sh) = refs
    c = lax.axis_index("c")
    s = lax.axis_index("s")
    wid = c * NS + s

    def _zrow(r, carry):
        for k in range(FDIM // 16):
            rows_a[r, pl.ds(k * 16, 16)] = jnp.zeros((16,), jnp.float32)
        return carry

    lax.fori_loop(0, ECHUNK, _zrow, 0)
    for t in range(RPT // ECHUNK):
        pltpu.sync_copy(rows_a, acc_sh.at[pl.ds(s * RPT + t * ECHUNK, ECHUNK)])
    if coef:
        def _zw(i, carry):
            wbuf_a[pl.ds(i * 16, 16)] = jnp.zeros((16,), jnp.float32)
            return carry

        lax.fori_loop(0, ECHUNK // 16, _zw, 0)
        for t in range(RPT // ECHUNK):
            pltpu.sync_copy(wbuf_a,
                            c_sh.at[pl.ds(s * RPT + t * ECHUNK, ECHUNK)])
    plsc.subcore_barrier()

    def _gather(j, rbuf, rsem, wbuf, wsem):
        pltpu.async_copy(table_hbm.at[src_v.at[j]], rbuf, rsem)
        if coef:
            pltpu.async_copy(w_hbm.at[dst_v.at[j]], wbuf, wsem)

    def _drain_scatter(j, rbuf, rsem, wbuf, wsem):
        pltpu.make_async_copy(table_hbm.at[src_v.at[j]], rbuf, rsem).wait()
        if coef:
            pltpu.make_async_copy(w_hbm.at[dst_v.at[j]], wbuf, wsem).wait()
            pltpu.sync_copy(wbuf, c_sh.at[src_v.at[j]], add=True)
        pltpu.sync_copy(rbuf, acc_sh.at[dst_v.at[j]], add=True)

    for half in range(2):
        pltpu.sync_copy(src_hbm.at[wid, pl.ds(half * HCHUNK, HCHUNK)], src_v)
        pltpu.sync_copy(dst_hbm.at[wid, pl.ds(half * HCHUNK, HCHUNK)], dst_v)
        _gather(0, rows_a, sem_a,
                wbuf_a if coef else None, sem_wa if coef else None)

        def _step(j2, carry):
            j = j2 * 2
            _gather(j + 1, rows_b, sem_b,
                    wbuf_b if coef else None, sem_wb if coef else None)
            _drain_scatter(j, rows_a, sem_a,
                           wbuf_a if coef else None,
                           sem_wa if coef else None)

            @pl.when(j + 2 < HCHUNK)
            def _():
                _gather(j + 2, rows_a, sem_a,
                        wbuf_a if coef else None,
                        sem_wa if coef else None)

            _drain_scatter(j + 1, rows_b, sem_b,
                           wbuf_b if coef else None,
                           sem_wb if coef else None)
            return carry

        lax.fori_loop(0, HCHUNK // 2, _step, 0)
    plsc.subcore_barrier()
    pltpu.sync_copy(acc_sh.at[pl.ds(s * RPT, RPT)],
                    part_out.at[c, pl.ds(s * RPT, RPT)])
    if coef:
        pltpu.sync_copy(c_sh.at[pl.ds(s * RPT, RPT)],
                        cpart_out.at[c, pl.ds(s * RPT, RPT)])


def _make_scatter(coef):
    part_type = jax.ShapeDtypeStruct((NC, NPAD, FDIM), jnp.float32)
    out_type = [part_type]
    scratch = [
        pltpu.VMEM((HCHUNK, ECHUNK), jnp.int32),
        pltpu.VMEM((HCHUNK, ECHUNK), jnp.int32),
        pltpu.VMEM((ECHUNK, FDIM), jnp.float32),
        pltpu.VMEM((ECHUNK, FDIM), jnp.float32),
        pltpu.SemaphoreType.DMA,
        pltpu.SemaphoreType.DMA,
    ]
    if coef:
        out_type.append(jax.ShapeDtypeStruct((NC, NPAD), jnp.float32))
        scratch += [
            pltpu.VMEM((ECHUNK,), jnp.float32),
            pltpu.VMEM((ECHUNK,), jnp.float32),
            pltpu.SemaphoreType.DMA,
            pltpu.SemaphoreType.DMA,
        ]
    scratch.append(pltpu.VMEM_SHARED((NPAD, FDIM), jnp.float32))
    if coef:
        scratch.append(pltpu.VMEM_SHARED((NPAD,), jnp.float32))
    return pl.kernel(
        functools.partial(_scatter_body, coef),
        out_type=out_type if coef else part_type,
        mesh=_mesh,
        scratch_types=scratch,
        compiler_params=_sc_params,
    )


_scatter_coef_call = _make_scatter(True)
_scatter_call = _make_scatter(False)


def _mm_body(x_ref, w_ref, o_ref):
    o_ref[...] = jnp.dot(x_ref[...], w_ref[...],
                         preferred_element_type=jnp.float32)


def _mm_call(x, W):
    return pl.pallas_call(
        _mm_body,
        grid=(GRID,),
        in_specs=[
            pl.BlockSpec((BLK, FDIM), lambda i: (i, 0)),
            pl.BlockSpec((FDIM, FDIM), lambda i: (0, 0)),
        ],
        out_specs=pl.BlockSpec((BLK, FDIM), lambda i: (i, 0)),
        out_shape=jax.ShapeDtypeStruct((NPAD, FDIM), jnp.float32),
    )(x, W)


def _k2_body(xw_ref, cntp_ref, y_ref, dinv_ref, wcol_ref):
    i = pl.program_id(0)
    cnt = cntp_ref[0] + cntp_ref[1]
    dinv = lax.rsqrt(cnt + 1.0)
    rows = i * BLK + lax.broadcasted_iota(jnp.int32, (BLK, 1), 0)
    wv = jnp.where(rows < N_NODES, 1.0 / jnp.maximum(cnt, 1.0), 0.0)
    y_ref[...] = xw_ref[...] * dinv
    dinv_ref[...] = dinv
    wcol_ref[...] = wv


def _k2_call(xw, cntp3):
    return pl.pallas_call(
        _k2_body,
        grid=(GRID,),
        in_specs=[
            pl.BlockSpec((BLK, FDIM), lambda i: (i, 0)),
            pl.BlockSpec((NC, BLK, 1), lambda i: (0, i, 0)),
        ],
        out_specs=[
            pl.BlockSpec((BLK, FDIM), lambda i: (i, 0)),
            pl.BlockSpec((BLK, 1), lambda i: (i, 0)),
            pl.BlockSpec((BLK, 1), lambda i: (i, 0)),
        ],
        out_shape=[
            jax.ShapeDtypeStruct((NPAD, FDIM), jnp.float32),
            jax.ShapeDtypeStruct((NPAD, 1), jnp.float32),
            jax.ShapeDtypeStruct((NPAD, 1), jnp.float32),
        ],
    )(xw, cntp3)


def _k4_body(p_ref, y_ref, dinv_ref, b_ref, h0_ref):
    t = (p_ref[0] + p_ref[1] + y_ref[...]) * dinv_ref[...]
    h0_ref[...] = jnp.maximum(t + b_ref[...], 0.0)


def _k4_call(parts, y, dinv_col, b_gcn2):
    return pl.pallas_call(
        _k4_body,
        grid=(GRID,),
        in_specs=[
            pl.BlockSpec((NC, BLK, FDIM), lambda i: (0, i, 0)),
            pl.BlockSpec((BLK, FDIM), lambda i: (i, 0)),
            pl.BlockSpec((BLK, 1), lambda i: (i, 0)),
            pl.BlockSpec((1, FDIM), lambda i: (0, 0)),
        ],
        out_specs=pl.BlockSpec((BLK, FDIM), lambda i: (i, 0)),
        out_shape=jax.ShapeDtypeStruct((NPAD, FDIM), jnp.float32),
    )(parts, y, dinv_col, b_gcn2)


def _k6_body(q_ref, r_ref, wcol_ref, cpart_ref, wl_ref, bl_ref,
             wl2, bl2, wr2, w0, b0, g0, be0, w1, b1, g1, be1,
             w2, b2, g2, be2, w3, b3, out_ref, s_ref):
    i = pl.program_id(0)
    agg = (q_ref[0] + q_ref[1]) * wcol_ref[...]
    h1 = (jnp.dot(agg, wl_ref[...], preferred_element_type=jnp.float32)
          + bl_ref[...] + r_ref[...])
    h1 = jnp.maximum(h1, 0.0)
    rows = i * BLK + lax.broadcasted_iota(jnp.int32, (BLK, 1), 0)
    h1m = jnp.where(rows < N_NODES, h1, 0.0)
    ccol = cpart_ref[0] + cpart_ref[1]
    s0 = jnp.sum(h1m, axis=0, keepdims=True)
    s1 = jnp.sum(ccol * h1m, axis=0, keepdims=True)
    blk = jnp.concatenate([s0, s1], axis=0)

    @pl.when(i == 0)
    def _():
        s_ref[...] = blk

    @pl.when(i > 0)
    def _():
        s_ref[...] += blk

    @pl.when(i == GRID - 1)
    def _():
        inv = 1.0 / jnp.sqrt(jnp.float32(1.0 + 1e-5))
        m_h1 = s_ref[0:1, :] * (1.0 / N_NODES)
        m_agg = s_ref[1:2, :] * (1.0 / N_NODES)
        p = (jnp.dot(m_agg, wl2[...], preferred_element_type=jnp.float32)
             + bl2[...]
             + jnp.dot(m_h1, wr2[...], preferred_element_type=jnp.float32))
        z = jnp.dot(p, w0[...], preferred_element_type=jnp.float32) + b0[...]
        z = jnp.tanh(z * inv * g0[...] + be0[...])
        z = jnp.dot(z, w1[...], preferred_element_type=jnp.float32) + b1[...]
        z = jnp.tanh(z * inv * g1[...] + be1[...])
        z = jnp.dot(z, w2[...], preferred_element_type=jnp.float32) + b2[...]
        z = jnp.tanh(z * inv * g2[...] + be2[...])
        z = jnp.dot(z, w3[...], preferred_element_type=jnp.float32) + b3[...]
        z = z - jnp.max(z, axis=1, keepdims=True)
        ez = jnp.exp(z)
        out_ref[...] = ez / jnp.sum(ez, axis=1, keepdims=True)


def _k6_call(parts2, R, w_col, cpart3, Wl1, bl1_2, *head):
    full = lambda a: pl.BlockSpec(a.shape, lambda i: tuple(0 for _ in a.shape))
    return pl.pallas_call(
        _k6_body,
        grid=(GRID,),
        in_specs=[
            pl.BlockSpec((NC, BLK, FDIM), lambda i: (0, i, 0)),
            pl.BlockSpec((BLK, FDIM), lambda i: (i, 0)),
            pl.BlockSpec((BLK, 1), lambda i: (i, 0)),
            pl.BlockSpec((NC, BLK, 1), lambda i: (0, i, 0)),
            pl.BlockSpec((FDIM, FDIM), lambda i: (0, 0)),
            pl.BlockSpec((1, FDIM), lambda i: (0, 0)),
        ] + [full(a) for a in head],
        out_specs=pl.BlockSpec((1, 10), lambda i: (0, 0)),
        out_shape=jax.ShapeDtypeStruct((1, 10), jnp.float32),
        scratch_shapes=[pltpu.VMEM((2, FDIM), jnp.float32)],
    )(parts2, R, w_col, cpart3, Wl1, bl1_2, *head)


def kernel(x, edge_index, W_gcn, b_gcn, Wl1, bl1, Wr1, Wl2, bl2, Wr2,
           W0, b0, g0, be0, W1, b1, g1, be1, W2, b2, g2, be2, W3, b3):
    f32 = jnp.float32
    n, d = x.shape
    x_pad = jnp.concatenate([x, jnp.zeros((NPAD - n, d), f32)], axis=0)

    src = edge_index[0]
    dst = edge_index[1]
    npad_e = EPAD - src.shape[0]
    pi = jnp.arange(npad_e, dtype=jnp.int32)
    src_p = jnp.concatenate([src, pi % N_NODES])
    dst_p = jnp.concatenate([dst, N_NODES + pi % TRASH])
    src3 = src_p.reshape(NW, NCHUNK, ECHUNK)
    dst3 = dst_p.reshape(NW, NCHUNK, ECHUNK)

    cntp = _count_call(dst3)
    xw = _mm_call(x_pad, W_gcn)
    y, dinv_col, w_col = _k2_call(xw, cntp.reshape(NC, NPAD, 1))
    parts, cpart = _scatter_coef_call(y, src3, dst3, w_col.reshape(NPAD))
    h0 = _k4_call(parts, y, dinv_col, b_gcn.reshape(1, FDIM))
    parts2 = _scatter_call(h0, src3, dst3)
    R = _mm_call(h0, Wr1)
    return _k6_call(parts2, R, w_col, cpart.reshape(NC, NPAD, 1),
                    Wl1, bl1.reshape(1, FDIM),
                    Wl2, bl2.reshape(1, FDIM), Wr2,
                    W0, b0.reshape(1, 200), g0.reshape(1, 200),
                    be0.reshape(1, 200),
                    W1, b1.reshape(1, 100), g1.reshape(1, 100),
                    be1.reshape(1, 100),
                    W2, b2.reshape(1, 50), g2.reshape(1, 50),
                    be2.reshape(1, 50),
                    W3, b3.reshape(1, 10))
